# baseline (speedup 1.0000x reference)
import functools

import jax
import jax.numpy as jnp
from jax import lax
from jax.experimental import pallas as pl
from jax.experimental.pallas import tpu as pltpu
from jax.experimental.pallas import tpu_sc as plsc

_VOCAB = 1000000
_BATCH = 4096
_SLOT_NUM = 26
_NNZ = 4
_EMB = 32
_PAD = _EMB + 1

_B = _BATCH * _SLOT_NUM * _NNZ
_NW = 32
_BPW = _B // _NW
_CHUNK = 128
_NCH = _BPW // _CHUNK

_BW = 512
_NBLK = _VOCAB // _BW
_REM = _VOCAB - _NBLK * _BW

_mesh = plsc.VectorSubcoreMesh(core_axis_name="c", subcore_axis_name="s")


def _transpose_block(src_v, dst_v, ncols):
    lanes = lax.iota(jnp.int32, 16)
    hi = lanes + 16

    @plsc.parallel_loop(0, ncols, unroll=8)
    def _col(j):
        col = jnp.full((16,), j, jnp.int32)
        v0 = plsc.load_gather(src_v, [lanes, col])
        v1 = plsc.load_gather(src_v, [hi, col])
        dst_v[pl.ds(j * _EMB, 16)] = v0
        dst_v[pl.ds(j * _EMB + 16, 16)] = v1


@functools.partial(
    pl.kernel,
    mesh=_mesh,
    out_type=jax.ShapeDtypeStruct((_VOCAB * _EMB,), jnp.float32),
    scratch_types=[
        pltpu.VMEM((_EMB, _BW + 9), jnp.float32),
        pltpu.VMEM((_EMB, _BW + 9), jnp.float32),
        pltpu.VMEM((_EMB, _REM), jnp.float32),
        pltpu.VMEM((_BW * _EMB,), jnp.float32),
        pltpu.SemaphoreType.DMA,
    ],
    compiler_params=pltpu.CompilerParams(
        use_tc_tiling_on_sc=True, needs_layout_passes=False
    ),
)
def _sc_transpose(tab_t, out_flat, blk_a, blk_b, blk_r, tr_v, sem):
    wid = lax.axis_index("s") * 2 + lax.axis_index("c")

    def fetch(c, buf):
        return pltpu.async_copy(
            tab_t.at[:, pl.ds(c * _BW, _BW)], buf.at[:, pl.ds(0, _BW)], sem
        )

    n_mine = (_NBLK - wid + _NW - 1) // _NW

    def emit(c, buf):
        _transpose_block(buf, tr_v, _BW)
        pltpu.sync_copy(
            tr_v, out_flat.at[pl.ds(c * (_BW * _EMB), _BW * _EMB)]
        )

    fetch(wid, blk_a).wait()

    def body(i, _):
        c = wid + (2 * i) * _NW
        c1 = jnp.where(2 * i + 1 < n_mine, c + _NW, wid)
        cp_b = fetch(c1, blk_b)
        emit(c, blk_a)
        cp_b.wait()
        c2 = jnp.where(2 * i + 2 < n_mine, c + 2 * _NW, wid)
        cp_a = fetch(c2, blk_a)

        @pl.when(2 * i + 1 < n_mine)
        def _odd():
            emit(c1, blk_b)

        cp_a.wait()
        return 0

    lax.fori_loop(0, (n_mine + 1) // 2, body, 0, unroll=False)

    @pl.when(wid == 0)
    def _rem():
        pltpu.async_copy(
            tab_t.at[:, pl.ds(_NBLK * _BW, _REM)], blk_r, sem
        ).wait()
        _transpose_block(blk_r, tr_v, _REM)
        pltpu.sync_copy(
            tr_v.at[pl.ds(0, _REM * _EMB)],
            out_flat.at[pl.ds(_NBLK * _BW * _EMB, _REM * _EMB)],
        )


@functools.partial(
    pl.kernel,
    mesh=_mesh,
    out_type=jax.ShapeDtypeStruct((_B, _EMB), jnp.float32),
    scratch_types=[
        pltpu.VMEM((_BPW,), jnp.int32),
        pltpu.VMEM((_CHUNK, _EMB), jnp.float32),
        pltpu.VMEM((_CHUNK, _EMB), jnp.float32),
        pltpu.SemaphoreType.DMA,
    ],
    compiler_params=pltpu.CompilerParams(use_tc_tiling_on_sc=False),
)
def _sc_gather(idx_hbm, table_hbm, out_hbm, idx_v, rows_a, rows_b, sem_g):
    wid = lax.axis_index("s") * 2 + lax.axis_index("c")
    base = wid * _BPW

    pltpu.sync_copy(idx_hbm.at[pl.ds(base, _BPW)], idx_v)

    def gather_start(i, buf):
        return pltpu.async_copy(
            table_hbm.at[idx_v.at[pl.ds(i * _CHUNK, _CHUNK)]],
            buf,
            sem_g,
        )

    def write_out(i, buf):
        pltpu.sync_copy(buf, out_hbm.at[pl.ds(base + i * _CHUNK, _CHUNK)])

    gather_start(0, rows_a).wait()

    def body(i, _):
        c0 = 2 * i
        cp_b = gather_start(jnp.where(c0 + 1 < _NCH, c0 + 1, 0), rows_b)
        write_out(c0, rows_a)
        cp_b.wait()
        cp_a = gather_start(jnp.where(c0 + 2 < _NCH, c0 + 2, 0), rows_a)

        @pl.when(c0 + 1 < _NCH)
        def _odd():
            write_out(c0 + 1, rows_b)

        cp_a.wait()
        return 0

    lax.fori_loop(0, (_NCH + 1) // 2, body, 0, unroll=False)


def kernel(inputs, table):
    flat = inputs.reshape(-1).astype(jnp.int32)
    tab_flat = _sc_transpose(table.T)
    tab_lin = tab_flat.reshape(_VOCAB, _EMB)
    out = _sc_gather(flat, tab_lin)
    return out.reshape(_BATCH, _SLOT_NUM, _NNZ, _EMB)


# speedup vs baseline: 1.0863x; 1.0863x over previous
import functools

import jax
import jax.numpy as jnp
from jax import lax
from jax.experimental import pallas as pl
from jax.experimental.pallas import tpu as pltpu
from jax.experimental.pallas import tpu_sc as plsc

_VOCAB = 1000000
_BATCH = 4096
_SLOT_NUM = 26
_NNZ = 4
_EMB = 32
_PAD = _EMB + 1

_B = _BATCH * _SLOT_NUM * _NNZ
_NW = 32
_BPW = _B // _NW
_CHUNK = 128
_NCH = _BPW // _CHUNK

_BW = 512
_NBLK = _VOCAB // _BW
_REM = _VOCAB - _NBLK * _BW

_mesh = plsc.VectorSubcoreMesh(core_axis_name="c", subcore_axis_name="s")


def _transpose_block(src_v, dst_v, ncols):
    lanes = lax.iota(jnp.int32, 16)
    hi = lanes + 16

    @plsc.parallel_loop(0, ncols, unroll=16)
    def _col(j):
        col = jnp.full((16,), j, jnp.int32)
        v0 = plsc.load_gather(src_v, [lanes, col])
        v1 = plsc.load_gather(src_v, [hi, col])
        dst_v[pl.ds(j * _EMB, 16)] = v0
        dst_v[pl.ds(j * _EMB + 16, 16)] = v1


@functools.partial(
    pl.kernel,
    mesh=_mesh,
    out_type=jax.ShapeDtypeStruct((_VOCAB * _EMB,), jnp.float32),
    scratch_types=[
        pltpu.VMEM((_EMB, _BW + 9), jnp.float32),
        pltpu.VMEM((_EMB, _BW + 9), jnp.float32),
        pltpu.VMEM((_EMB, _REM), jnp.float32),
        pltpu.VMEM((_BW * _EMB,), jnp.float32),
        pltpu.SemaphoreType.DMA,
    ],
    compiler_params=pltpu.CompilerParams(
        use_tc_tiling_on_sc=True, needs_layout_passes=False
    ),
)
def _sc_transpose(tab_t, out_flat, blk_a, blk_b, blk_r, tr_v, sem):
    wid = lax.axis_index("s") * 2 + lax.axis_index("c")

    def fetch(c, buf):
        return pltpu.async_copy(
            tab_t.at[:, pl.ds(c * _BW, _BW)], buf.at[:, pl.ds(0, _BW)], sem
        )

    n_mine = (_NBLK - wid + _NW - 1) // _NW

    def emit(c, buf):
        _transpose_block(buf, tr_v, _BW)
        pltpu.sync_copy(
            tr_v, out_flat.at[pl.ds(c * (_BW * _EMB), _BW * _EMB)]
        )

    fetch(wid, blk_a).wait()

    def body(i, _):
        c = wid + (2 * i) * _NW
        c1 = jnp.where(2 * i + 1 < n_mine, c + _NW, wid)
        cp_b = fetch(c1, blk_b)
        emit(c, blk_a)
        cp_b.wait()
        c2 = jnp.where(2 * i + 2 < n_mine, c + 2 * _NW, wid)
        cp_a = fetch(c2, blk_a)

        @pl.when(2 * i + 1 < n_mine)
        def _odd():
            emit(c1, blk_b)

        cp_a.wait()
        return 0

    lax.fori_loop(0, (n_mine + 1) // 2, body, 0, unroll=False)

    @pl.when(wid == 0)
    def _rem():
        pltpu.async_copy(
            tab_t.at[:, pl.ds(_NBLK * _BW, _REM)], blk_r, sem
        ).wait()
        _transpose_block(blk_r, tr_v, _REM)
        pltpu.sync_copy(
            tr_v.at[pl.ds(0, _REM * _EMB)],
            out_flat.at[pl.ds(_NBLK * _BW * _EMB, _REM * _EMB)],
        )


@functools.partial(
    pl.kernel,
    mesh=_mesh,
    out_type=jax.ShapeDtypeStruct((_B, _EMB), jnp.float32),
    scratch_types=[
        pltpu.VMEM((_BPW,), jnp.int32),
        pltpu.VMEM((_CHUNK, _EMB), jnp.float32),
        pltpu.VMEM((_CHUNK, _EMB), jnp.float32),
        pltpu.SemaphoreType.DMA,
    ],
    compiler_params=pltpu.CompilerParams(use_tc_tiling_on_sc=False),
)
def _sc_gather(idx_hbm, table_hbm, out_hbm, idx_v, rows_a, rows_b, sem_g):
    wid = lax.axis_index("s") * 2 + lax.axis_index("c")
    base = wid * _BPW

    pltpu.sync_copy(idx_hbm.at[pl.ds(base, _BPW)], idx_v)

    def gather_start(i, buf):
        return pltpu.async_copy(
            table_hbm.at[idx_v.at[pl.ds(i * _CHUNK, _CHUNK)]],
            buf,
            sem_g,
        )

    def write_out(i, buf):
        pltpu.sync_copy(buf, out_hbm.at[pl.ds(base + i * _CHUNK, _CHUNK)])

    gather_start(0, rows_a).wait()

    def body(i, _):
        c0 = 2 * i
        cp_b = gather_start(jnp.where(c0 + 1 < _NCH, c0 + 1, 0), rows_b)
        write_out(c0, rows_a)
        cp_b.wait()
        cp_a = gather_start(jnp.where(c0 + 2 < _NCH, c0 + 2, 0), rows_a)

        @pl.when(c0 + 1 < _NCH)
        def _odd():
            write_out(c0 + 1, rows_b)

        cp_a.wait()
        return 0

    lax.fori_loop(0, (_NCH + 1) // 2, body, 0, unroll=False)


def kernel(inputs, table):
    flat = inputs.reshape(-1).astype(jnp.int32)
    tab_flat = _sc_transpose(table.T)
    tab_lin = tab_flat.reshape(_VOCAB, _EMB)
    out = _sc_gather(flat, tab_lin)
    return out.reshape(_BATCH, _SLOT_NUM, _NNZ, _EMB)
